# baseline (speedup 1.0000x reference)
import functools

import jax
import jax.numpy as jnp
from jax import lax
from jax.experimental import pallas as pl
from jax.experimental.pallas import tpu as pltpu
from jax.experimental.pallas import tpu_sc as plsc

S, D, H, E, K, DFF, CAP = 2048, 1024, 16, 8, 2, 2048, 768
DH = D // H
TK = S * K
NSLOT = E * CAP
BS = 512
EBS = 256
EPS = 1e-5
LANES = 128


def _qkv_route_body(x_ref, wq_ref, wk_ref, wv_ref, wr_ref,
                    q_ref, k_ref, v_ref, oh_ref, meta_ref):
    xf32 = x_ref[...]
    x = xf32.astype(jnp.bfloat16)
    wqb = wq_ref[...].astype(jnp.bfloat16)
    wkb = wk_ref[...].astype(jnp.bfloat16)
    wvb = wv_ref[...].astype(jnp.bfloat16)
    q_ref[...] = jnp.dot(x, wqb, preferred_element_type=jnp.float32)
    k_ref[...] = jnp.dot(x, wkb, preferred_element_type=jnp.float32)
    v_ref[...] = jnp.dot(x, wvb, preferred_element_type=jnp.float32)
    logits = jnp.dot(xf32, wr_ref[...], preferred_element_type=jnp.float32)
    lane = jax.lax.broadcasted_iota(jnp.int32, (BS, LANES), 1)
    l = jnp.where(lane < E, logits, jnp.float32(-1e30))
    m = jnp.max(l, axis=1, keepdims=True)
    ex = jnp.exp(l - m)
    p = ex / jnp.sum(ex, axis=1, keepdims=True)
    m1 = jnp.max(p, axis=1, keepdims=True)
    i1 = jnp.min(jnp.where(p == m1, lane, jnp.int32(LANES)), axis=1, keepdims=True)
    p2 = jnp.where(lane == i1, jnp.float32(-1.0), p)
    m2 = jnp.max(p2, axis=1, keepdims=True)
    i2 = jnp.min(jnp.where(p2 == m2, lane, jnp.int32(LANES)), axis=1, keepdims=True)
    den = m1 + m2
    w0 = m1 / den
    w1 = m2 / den
    oh_ref[...] = ((lane == i1) | (lane == i2)).astype(jnp.bfloat16)
    i1f = i1.astype(jnp.float32)
    i2f = i2.astype(jnp.float32)
    meta_ref[...] = jnp.where(lane == 0, i1f,
                    jnp.where(lane == 1, i2f,
                    jnp.where(lane == 2, w0,
                    jnp.where(lane == 3, w1, jnp.float32(0.0)))))


def _qkv_route(xf, wq, wk, wv, wr_pad):
    w_spec = pl.BlockSpec((D, D), lambda i: (0, 0))
    r_spec = pl.BlockSpec((BS, D), lambda i: (i, 0))
    l_spec = pl.BlockSpec((BS, LANES), lambda i: (i, 0))
    return pl.pallas_call(
        _qkv_route_body,
        grid=(S // BS,),
        in_specs=[r_spec, w_spec, w_spec, w_spec,
                  pl.BlockSpec((D, LANES), lambda i: (0, 0))],
        out_specs=[r_spec, r_spec, r_spec, l_spec, l_spec],
        out_shape=[jax.ShapeDtypeStruct((S, D), jnp.float32)] * 3
        + [jax.ShapeDtypeStruct((S, LANES), jnp.bfloat16),
           jax.ShapeDtypeStruct((S, LANES), jnp.float32)],
    )(xf, wq, wk, wv, wr_pad)


def _attn_body(q_ref, kt_ref, v_ref, o_ref):
    qb = pl.program_id(1)
    qb16 = q_ref[0].astype(jnp.bfloat16)
    kt16 = kt_ref[0].astype(jnp.bfloat16)
    s = jnp.dot(qb16, kt16, preferred_element_type=jnp.float32)
    s = s * 0.125
    row = jax.lax.broadcasted_iota(jnp.int32, (BS, S), 0) + qb * BS
    col = jax.lax.broadcasted_iota(jnp.int32, (BS, S), 1)
    s = jnp.where(col <= row, s, jnp.float32(-1e30))
    m = jnp.max(s, axis=1, keepdims=True)
    e = jnp.exp(s - m)
    p = (e / jnp.sum(e, axis=1, keepdims=True)).astype(jnp.bfloat16)
    v16 = v_ref[0].astype(jnp.bfloat16)
    o_ref[0] = jnp.dot(p, v16, preferred_element_type=jnp.float32)


def _attn(qh, kth, vh):
    return pl.pallas_call(
        _attn_body,
        grid=(H, S // BS),
        in_specs=[
            pl.BlockSpec((1, BS, DH), lambda h, qb: (h, qb, 0)),
            pl.BlockSpec((1, DH, S), lambda h, qb: (h, 0, 0)),
            pl.BlockSpec((1, S, DH), lambda h, qb: (h, 0, 0)),
        ],
        out_specs=pl.BlockSpec((1, BS, DH), lambda h, qb: (h, qb, 0)),
        out_shape=jax.ShapeDtypeStruct((H, S, DH), jnp.float32),
    )(qh, kth, vh)


def _rmsnorm(v, w_row):
    ms = jnp.mean(v * v, axis=1, keepdims=True)
    return v * jax.lax.rsqrt(ms + EPS) * w_row


def _oproj_body(o_ref, wo_ref, x_ref, nw_ref, h_ref):
    ob = o_ref[...].astype(jnp.bfloat16)
    wob = wo_ref[...].astype(jnp.bfloat16)
    prj = jnp.dot(ob, wob, preferred_element_type=jnp.float32)
    h_ref[...] = x_ref[...] + _rmsnorm(prj, nw_ref[...])


def _oproj(o, wo, xf, nw):
    return pl.pallas_call(
        _oproj_body,
        grid=(S // BS,),
        in_specs=[
            pl.BlockSpec((BS, D), lambda i: (i, 0)),
            pl.BlockSpec((D, D), lambda i: (0, 0)),
            pl.BlockSpec((BS, D), lambda i: (i, 0)),
            pl.BlockSpec((1, D), lambda i: (0, 0)),
        ],
        out_specs=pl.BlockSpec((BS, D), lambda i: (i, 0)),
        out_shape=jax.ShapeDtypeStruct((S, D), jnp.float32),
    )(o, wo, xf, nw)


def _ffn_body(h_ref, w1_ref, w3_ref, w2_ref, nw_ref, out_ref):
    h = h_ref[...]
    hb = h.astype(jnp.bfloat16)
    a = jnp.dot(hb, w1_ref[...].astype(jnp.bfloat16), preferred_element_type=jnp.float32)
    b = jnp.dot(hb, w3_ref[...].astype(jnp.bfloat16), preferred_element_type=jnp.float32)
    g = ((a / (1.0 + jnp.exp(-a))) * b).astype(jnp.bfloat16)
    ff = jnp.dot(g, w2_ref[...].astype(jnp.bfloat16), preferred_element_type=jnp.float32)
    out_ref[...] = h + _rmsnorm(ff, nw_ref[...])


def _ffn(h, w1, w3, w2, nw):
    return pl.pallas_call(
        _ffn_body,
        grid=(S // BS,),
        in_specs=[
            pl.BlockSpec((BS, D), lambda i: (i, 0)),
            pl.BlockSpec((D, DFF), lambda i: (0, 0)),
            pl.BlockSpec((D, DFF), lambda i: (0, 0)),
            pl.BlockSpec((DFF, D), lambda i: (0, 0)),
            pl.BlockSpec((1, D), lambda i: (0, 0)),
        ],
        out_specs=pl.BlockSpec((BS, D), lambda i: (i, 0)),
        out_shape=jax.ShapeDtypeStruct((S, D), jnp.float32),
    )(h, w1, w3, w2, nw)


def _route_pos_body(oh_ref, meta_ref, idx_ref, w_ref, cnt_ref):
    rb = pl.program_id(0)
    row = jax.lax.broadcasted_iota(jnp.int32, (BS, S), 0) + rb * BS
    col = jax.lax.broadcasted_iota(jnp.int32, (BS, S), 1)
    ltri = (col < row).astype(jnp.bfloat16)
    sx = jnp.dot(ltri, oh_ref[...], preferred_element_type=jnp.float32)
    lane = jax.lax.broadcasted_iota(jnp.int32, (BS, LANES), 1)
    lanef = lane.astype(jnp.float32)
    i1f = meta_ref[:, 0:1]
    i2f = meta_ref[:, 1:2]
    w0 = meta_ref[:, 2:3]
    w1 = meta_ref[:, 3:4]
    pos1 = jnp.sum(jnp.where(lanef == i1f, sx, 0.0), axis=1, keepdims=True).astype(jnp.int32)
    pos2 = jnp.sum(jnp.where(lanef == i2f, sx, 0.0), axis=1, keepdims=True).astype(jnp.int32)
    k1 = pos1 < CAP
    k2 = pos2 < CAP
    g1 = i1f.astype(jnp.int32) * CAP + jnp.minimum(pos1, CAP - 1)
    g2 = i2f.astype(jnp.int32) * CAP + jnp.minimum(pos2, CAP - 1)
    s1 = jnp.where(k1, g1, NSLOT)
    s2 = jnp.where(k2, g2, NSLOT)
    idx = jnp.where(lane == 0, s1,
          jnp.where(lane == 1, s2,
          jnp.where(lane == 2, g1,
          jnp.where(lane == 3, g2, jnp.int32(0)))))
    w = jnp.where(lane == 0, w0 * k1.astype(jnp.float32),
        jnp.where(lane == 1, w1 * k2.astype(jnp.float32), jnp.float32(0.0)))
    idx_ref[...] = idx
    w_ref[...] = w

    @pl.when(rb == S // BS - 1)
    def _():
        tot = sx[BS - 1:BS, :] + oh_ref[S - 1:S, :].astype(jnp.float32)
        cnt_ref[...] = jnp.broadcast_to(tot, (8, LANES)).astype(jnp.int32)


def _route_pos(oh, meta):
    return pl.pallas_call(
        _route_pos_body,
        grid=(S // BS,),
        in_specs=[
            pl.BlockSpec((S, LANES), lambda i: (0, 0)),
            pl.BlockSpec((BS, LANES), lambda i: (i, 0)),
        ],
        out_specs=[
            pl.BlockSpec((BS, LANES), lambda i: (i, 0)),
            pl.BlockSpec((BS, LANES), lambda i: (i, 0)),
            pl.BlockSpec((8, LANES), lambda i: (0, 0)),
        ],
        out_shape=[
            jax.ShapeDtypeStruct((S, LANES), jnp.int32),
            jax.ShapeDtypeStruct((S, LANES), jnp.float32),
            jax.ShapeDtypeStruct((8, LANES), jnp.int32),
        ],
    )(oh, meta)


def _experts_flat_body(cnt_ref, b_ref, w1_ref, w2_ref, o_ref):
    e = pl.program_id(0)
    c = pl.program_id(1)

    @pl.when(c * EBS < cnt_ref[e])
    def _():
        x = b_ref[...].astype(jnp.bfloat16)
        a = jnp.dot(x, w1_ref[0].astype(jnp.bfloat16), preferred_element_type=jnp.float32)
        hdn = jax.nn.gelu(a).astype(jnp.bfloat16)
        o_ref[...] = jnp.dot(hdn, w2_ref[0].astype(jnp.bfloat16),
                             preferred_element_type=jnp.float32)


def _experts(counts, buf_flat, we1, we2):
    return pl.pallas_call(
        _experts_flat_body,
        grid=(E, CAP // EBS),
        in_specs=[
            pl.BlockSpec(memory_space=pltpu.SMEM),
            pl.BlockSpec((EBS, D), lambda e, c: (e * (CAP // EBS) + c, 0)),
            pl.BlockSpec((1, D, DFF), lambda e, c: (e, 0, 0)),
            pl.BlockSpec((1, DFF, D), lambda e, c: (e, 0, 0)),
        ],
        out_specs=pl.BlockSpec((EBS, D), lambda e, c: (e * (CAP // EBS) + c, 0)),
        out_shape=jax.ShapeDtypeStruct((NSLOT, D), jnp.float32),
    )(counts, buf_flat, we1, we2)


def _combine_body(yt_ref, w_ref, dense_ref, nw_ref, out_ref):
    y = yt_ref[:, :D] * w_ref[:, 0:1] + yt_ref[:, D:] * w_ref[:, 1:2]
    out_ref[...] = dense_ref[...] + _rmsnorm(y, nw_ref[...])


def _combine(yt2, wcomb, dense, nw):
    return pl.pallas_call(
        _combine_body,
        grid=(S // BS,),
        in_specs=[
            pl.BlockSpec((BS, 2 * D), lambda i: (i, 0)),
            pl.BlockSpec((BS, LANES), lambda i: (i, 0)),
            pl.BlockSpec((BS, D), lambda i: (i, 0)),
            pl.BlockSpec((1, D), lambda i: (0, 0)),
        ],
        out_specs=pl.BlockSpec((BS, D), lambda i: (i, 0)),
        out_shape=jax.ShapeDtypeStruct((S, D), jnp.float32),
    )(yt2, wcomb, dense, nw)


_SC_MESH = plsc.VectorSubcoreMesh(core_axis_name="c", subcore_axis_name="s")
_NC = 2
_NW = 32


NBUF = NSLOT + 8
_TPW = S // _NW


@functools.partial(
    pl.kernel,
    mesh=_SC_MESH,
    out_type=jax.ShapeDtypeStruct((NBUF, D), jnp.float32),
    scratch_types=[
        pltpu.VMEM((_TPW, D), jnp.float32),
        pltpu.VMEM((_TPW,), jnp.int32),
        pltpu.VMEM((_TPW,), jnp.int32),
        pltpu.SemaphoreType.DMA,
        pltpu.SemaphoreType.DMA,
    ],
)
def _sc_dispatch(xf_hbm, idxa_hbm, idxb_hbm, buf_hbm, rows_v, ia_v, ib_v, sa, sb):
    wid = lax.axis_index("s") * _NC + lax.axis_index("c")
    base = wid * _TPW
    pltpu.sync_copy(idxa_hbm.at[pl.ds(base, _TPW)], ia_v)
    pltpu.sync_copy(idxb_hbm.at[pl.ds(base, _TPW)], ib_v)
    pltpu.sync_copy(xf_hbm.at[pl.ds(base, _TPW)], rows_v)
    cpa = pltpu.async_copy(rows_v, buf_hbm.at[ia_v], sa)
    cpb = pltpu.async_copy(rows_v, buf_hbm.at[ib_v], sb)
    cpa.wait()
    cpb.wait()


def _make_sc_gather(n_rows):
    per_w = n_rows // _NW
    chunk = 32
    n_chunks = per_w // chunk

    @functools.partial(
        pl.kernel,
        mesh=_SC_MESH,
        out_type=jax.ShapeDtypeStruct((n_rows, D), jnp.float32),
        scratch_types=[pltpu.VMEM((per_w,), jnp.int32)]
        + [pltpu.VMEM((chunk, D), jnp.float32)] * 2
        + [pltpu.SemaphoreType.DMA] * 2,
    )
    def gather(table_hbm, idx_hbm, out_hbm, idx_v, buf0, buf1, sem0, sem1):
        bufs = (buf0, buf1)
        sems = (sem0, sem1)
        wid = lax.axis_index("s") * _NC + lax.axis_index("c")
        base_w = wid * per_w
        pltpu.sync_copy(idx_hbm.at[pl.ds(base_w, per_w)], idx_v)

        def start(c):
            return pltpu.async_copy(
                table_hbm.at[idx_v.at[pl.ds(c * chunk, chunk)]],
                bufs[c % 2], sems[c % 2])

        cps = [start(0), None]
        for c in range(n_chunks):
            if c + 1 < n_chunks:
                cps[(c + 1) % 2] = start(c + 1)
            cps[c % 2].wait()
            pltpu.sync_copy(bufs[c % 2], out_hbm.at[pl.ds(base_w + c * chunk, chunk)])

    return gather


_sc_gather_combine = _make_sc_gather(TK)


def kernel(x, wq, wk, wv, wo, attn_norm_w, w1, w2, w3, ffn_norm_w,
           wr, we1, we2, moe_norm_w):
    xf = x.reshape(S, D)

    wr_pad = jnp.pad(wr, ((0, 0), (0, LANES - E)))
    q, k, v, oh, meta = _qkv_route(xf, wq, wk, wv, wr_pad)
    qh = q.reshape(S, H, DH).transpose(1, 0, 2)
    kth = k.reshape(S, H, DH).transpose(1, 2, 0)
    vh = v.reshape(S, H, DH).transpose(1, 0, 2)
    o = _attn(qh, kth, vh).transpose(1, 0, 2).reshape(S, D)
    h = _oproj(o, wo, xf, attn_norm_w.reshape(1, D))
    dense = _ffn(h, w1, w3, w2, ffn_norm_w.reshape(1, D))

    idxk, wcomb, cnts = _route_pos(oh, meta)
    gath_idx = idxk[:, K:2 * K].reshape(TK)
    counts = cnts[0, :E]

    buf = _sc_dispatch(xf, idxk[:, 0], idxk[:, 1])
    eout = _experts(counts, buf, we1, we2)
    yt = _sc_gather_combine(eout, gath_idx)

    out = _combine(yt.reshape(S, 2 * D), wcomb, dense, moe_norm_w.reshape(1, D))
    return out.reshape(1, S, D)


# speedup vs baseline: 1.0181x; 1.0181x over previous
import functools

import jax
import jax.numpy as jnp
from jax import lax
from jax.experimental import pallas as pl
from jax.experimental.pallas import tpu as pltpu
from jax.experimental.pallas import tpu_sc as plsc

S, D, H, E, K, DFF, CAP = 2048, 1024, 16, 8, 2, 2048, 768
DH = D // H
TK = S * K
NSLOT = E * CAP
BS = 256
EBS = 256
EPS = 1e-5
LANES = 128


def _qkv_route_body(x_ref, wq_ref, wk_ref, wv_ref, wr_ref,
                    q_ref, k_ref, v_ref, oh_ref, meta_ref):
    xf32 = x_ref[...]
    x = xf32.astype(jnp.bfloat16)
    wqb = wq_ref[...].astype(jnp.bfloat16)
    wkb = wk_ref[...].astype(jnp.bfloat16)
    wvb = wv_ref[...].astype(jnp.bfloat16)
    q_ref[...] = jnp.dot(x, wqb, preferred_element_type=jnp.float32)
    k_ref[...] = jnp.dot(x, wkb, preferred_element_type=jnp.float32)
    v_ref[...] = jnp.dot(x, wvb, preferred_element_type=jnp.float32)
    logits = jnp.dot(xf32, wr_ref[...], preferred_element_type=jnp.float32)
    lane = jax.lax.broadcasted_iota(jnp.int32, (BS, LANES), 1)
    l = jnp.where(lane < E, logits, jnp.float32(-1e30))
    m = jnp.max(l, axis=1, keepdims=True)
    ex = jnp.exp(l - m)
    p = ex / jnp.sum(ex, axis=1, keepdims=True)
    m1 = jnp.max(p, axis=1, keepdims=True)
    i1 = jnp.min(jnp.where(p == m1, lane, jnp.int32(LANES)), axis=1, keepdims=True)
    p2 = jnp.where(lane == i1, jnp.float32(-1.0), p)
    m2 = jnp.max(p2, axis=1, keepdims=True)
    i2 = jnp.min(jnp.where(p2 == m2, lane, jnp.int32(LANES)), axis=1, keepdims=True)
    den = m1 + m2
    w0 = m1 / den
    w1 = m2 / den
    oh_ref[...] = ((lane == i1) | (lane == i2)).astype(jnp.bfloat16)
    i1f = i1.astype(jnp.float32)
    i2f = i2.astype(jnp.float32)
    meta_ref[...] = jnp.where(lane == 0, i1f,
                    jnp.where(lane == 1, i2f,
                    jnp.where(lane == 2, w0,
                    jnp.where(lane == 3, w1, jnp.float32(0.0)))))


def _qkv_route(xf, wq, wk, wv, wr_pad):
    w_spec = pl.BlockSpec((D, D), lambda i: (0, 0))
    r_spec = pl.BlockSpec((BS, D), lambda i: (i, 0))
    l_spec = pl.BlockSpec((BS, LANES), lambda i: (i, 0))
    return pl.pallas_call(
        _qkv_route_body,
        grid=(S // BS,),
        in_specs=[r_spec, w_spec, w_spec, w_spec,
                  pl.BlockSpec((D, LANES), lambda i: (0, 0))],
        out_specs=[r_spec, r_spec, r_spec, l_spec, l_spec],
        out_shape=[jax.ShapeDtypeStruct((S, D), jnp.float32)] * 3
        + [jax.ShapeDtypeStruct((S, LANES), jnp.bfloat16),
           jax.ShapeDtypeStruct((S, LANES), jnp.float32)],
    )(xf, wq, wk, wv, wr_pad)


def _attn_body(q_ref, kt_ref, v_ref, o_ref):
    qb = pl.program_id(1)
    qb16 = q_ref[0].astype(jnp.bfloat16)
    kt16 = kt_ref[0].astype(jnp.bfloat16)
    s = jnp.dot(qb16, kt16, preferred_element_type=jnp.float32)
    s = s * 0.125
    row = jax.lax.broadcasted_iota(jnp.int32, (BS, S), 0) + qb * BS
    col = jax.lax.broadcasted_iota(jnp.int32, (BS, S), 1)
    s = jnp.where(col <= row, s, jnp.float32(-1e30))
    m = jnp.max(s, axis=1, keepdims=True)
    e = jnp.exp(s - m)
    p = (e / jnp.sum(e, axis=1, keepdims=True)).astype(jnp.bfloat16)
    v16 = v_ref[0].astype(jnp.bfloat16)
    o_ref[0] = jnp.dot(p, v16, preferred_element_type=jnp.float32)


def _attn(qh, kth, vh):
    return pl.pallas_call(
        _attn_body,
        grid=(H, S // BS),
        in_specs=[
            pl.BlockSpec((1, BS, DH), lambda h, qb: (h, qb, 0)),
            pl.BlockSpec((1, DH, S), lambda h, qb: (h, 0, 0)),
            pl.BlockSpec((1, S, DH), lambda h, qb: (h, 0, 0)),
        ],
        out_specs=pl.BlockSpec((1, BS, DH), lambda h, qb: (h, qb, 0)),
        out_shape=jax.ShapeDtypeStruct((H, S, DH), jnp.float32),
    )(qh, kth, vh)


def _rmsnorm(v, w_row):
    ms = jnp.mean(v * v, axis=1, keepdims=True)
    return v * jax.lax.rsqrt(ms + EPS) * w_row


def _oproj_body(o_ref, wo_ref, x_ref, nw_ref, h_ref):
    ob = o_ref[...].astype(jnp.bfloat16)
    wob = wo_ref[...].astype(jnp.bfloat16)
    prj = jnp.dot(ob, wob, preferred_element_type=jnp.float32)
    h_ref[...] = x_ref[...] + _rmsnorm(prj, nw_ref[...])


def _oproj(o, wo, xf, nw):
    return pl.pallas_call(
        _oproj_body,
        grid=(S // BS,),
        in_specs=[
            pl.BlockSpec((BS, D), lambda i: (i, 0)),
            pl.BlockSpec((D, D), lambda i: (0, 0)),
            pl.BlockSpec((BS, D), lambda i: (i, 0)),
            pl.BlockSpec((1, D), lambda i: (0, 0)),
        ],
        out_specs=pl.BlockSpec((BS, D), lambda i: (i, 0)),
        out_shape=jax.ShapeDtypeStruct((S, D), jnp.float32),
    )(o, wo, xf, nw)


def _ffn_body(h_ref, w1_ref, w3_ref, w2_ref, nw_ref, out_ref):
    h = h_ref[...]
    hb = h.astype(jnp.bfloat16)
    a = jnp.dot(hb, w1_ref[...].astype(jnp.bfloat16), preferred_element_type=jnp.float32)
    b = jnp.dot(hb, w3_ref[...].astype(jnp.bfloat16), preferred_element_type=jnp.float32)
    g = ((a / (1.0 + jnp.exp(-a))) * b).astype(jnp.bfloat16)
    ff = jnp.dot(g, w2_ref[...].astype(jnp.bfloat16), preferred_element_type=jnp.float32)
    out_ref[...] = h + _rmsnorm(ff, nw_ref[...])


def _ffn(h, w1, w3, w2, nw):
    return pl.pallas_call(
        _ffn_body,
        grid=(S // BS,),
        in_specs=[
            pl.BlockSpec((BS, D), lambda i: (i, 0)),
            pl.BlockSpec((D, DFF), lambda i: (0, 0)),
            pl.BlockSpec((D, DFF), lambda i: (0, 0)),
            pl.BlockSpec((DFF, D), lambda i: (0, 0)),
            pl.BlockSpec((1, D), lambda i: (0, 0)),
        ],
        out_specs=pl.BlockSpec((BS, D), lambda i: (i, 0)),
        out_shape=jax.ShapeDtypeStruct((S, D), jnp.float32),
    )(h, w1, w3, w2, nw)


def _route_pos_body(oh_ref, meta_ref, idx_ref, w_ref, cnt_ref):
    rb = pl.program_id(0)
    row = jax.lax.broadcasted_iota(jnp.int32, (BS, S), 0) + rb * BS
    col = jax.lax.broadcasted_iota(jnp.int32, (BS, S), 1)
    ltri = (col < row).astype(jnp.bfloat16)
    sx = jnp.dot(ltri, oh_ref[...], preferred_element_type=jnp.float32)
    lane = jax.lax.broadcasted_iota(jnp.int32, (BS, LANES), 1)
    lanef = lane.astype(jnp.float32)
    i1f = meta_ref[:, 0:1]
    i2f = meta_ref[:, 1:2]
    w0 = meta_ref[:, 2:3]
    w1 = meta_ref[:, 3:4]
    pos1 = jnp.sum(jnp.where(lanef == i1f, sx, 0.0), axis=1, keepdims=True).astype(jnp.int32)
    pos2 = jnp.sum(jnp.where(lanef == i2f, sx, 0.0), axis=1, keepdims=True).astype(jnp.int32)
    k1 = pos1 < CAP
    k2 = pos2 < CAP
    g1 = i1f.astype(jnp.int32) * CAP + jnp.minimum(pos1, CAP - 1)
    g2 = i2f.astype(jnp.int32) * CAP + jnp.minimum(pos2, CAP - 1)
    s1 = jnp.where(k1, g1, NSLOT)
    s2 = jnp.where(k2, g2, NSLOT)
    idx = jnp.where(lane == 0, s1,
          jnp.where(lane == 1, s2,
          jnp.where(lane == 2, g1,
          jnp.where(lane == 3, g2, jnp.int32(0)))))
    w = jnp.where(lane == 0, w0 * k1.astype(jnp.float32),
        jnp.where(lane == 1, w1 * k2.astype(jnp.float32), jnp.float32(0.0)))
    idx_ref[...] = idx
    w_ref[...] = w

    @pl.when(rb == S // BS - 1)
    def _():
        tot = sx[BS - 1:BS, :] + oh_ref[S - 1:S, :].astype(jnp.float32)
        cnt_ref[...] = jnp.broadcast_to(tot, (8, LANES)).astype(jnp.int32)


def _route_pos(oh, meta):
    return pl.pallas_call(
        _route_pos_body,
        grid=(S // BS,),
        in_specs=[
            pl.BlockSpec((S, LANES), lambda i: (0, 0)),
            pl.BlockSpec((BS, LANES), lambda i: (i, 0)),
        ],
        out_specs=[
            pl.BlockSpec((BS, LANES), lambda i: (i, 0)),
            pl.BlockSpec((BS, LANES), lambda i: (i, 0)),
            pl.BlockSpec((8, LANES), lambda i: (0, 0)),
        ],
        out_shape=[
            jax.ShapeDtypeStruct((S, LANES), jnp.int32),
            jax.ShapeDtypeStruct((S, LANES), jnp.float32),
            jax.ShapeDtypeStruct((8, LANES), jnp.int32),
        ],
    )(oh, meta)


def _experts_flat_body(cnt_ref, b_ref, w1_ref, w2_ref, o_ref):
    e = pl.program_id(0)
    c = pl.program_id(1)

    @pl.when(c * EBS < cnt_ref[e])
    def _():
        x = b_ref[...].astype(jnp.bfloat16)
        a = jnp.dot(x, w1_ref[0].astype(jnp.bfloat16), preferred_element_type=jnp.float32)
        hdn = jax.nn.gelu(a).astype(jnp.bfloat16)
        o_ref[...] = jnp.dot(hdn, w2_ref[0].astype(jnp.bfloat16),
                             preferred_element_type=jnp.float32)


def _experts(counts, buf_flat, we1, we2):
    return pl.pallas_call(
        _experts_flat_body,
        grid=(E, CAP // EBS),
        in_specs=[
            pl.BlockSpec(memory_space=pltpu.SMEM),
            pl.BlockSpec((EBS, D), lambda e, c: (e * (CAP // EBS) + c, 0)),
            pl.BlockSpec((1, D, DFF), lambda e, c: (e, 0, 0)),
            pl.BlockSpec((1, DFF, D), lambda e, c: (e, 0, 0)),
        ],
        out_specs=pl.BlockSpec((EBS, D), lambda e, c: (e * (CAP // EBS) + c, 0)),
        out_shape=jax.ShapeDtypeStruct((NSLOT, D), jnp.float32),
    )(counts, buf_flat, we1, we2)


def _combine_body(yt_ref, w_ref, dense_ref, nw_ref, out_ref):
    y = yt_ref[:, :D] * w_ref[:, 0:1] + yt_ref[:, D:] * w_ref[:, 1:2]
    out_ref[...] = dense_ref[...] + _rmsnorm(y, nw_ref[...])


def _combine(yt2, wcomb, dense, nw):
    return pl.pallas_call(
        _combine_body,
        grid=(S // BS,),
        in_specs=[
            pl.BlockSpec((BS, 2 * D), lambda i: (i, 0)),
            pl.BlockSpec((BS, LANES), lambda i: (i, 0)),
            pl.BlockSpec((BS, D), lambda i: (i, 0)),
            pl.BlockSpec((1, D), lambda i: (0, 0)),
        ],
        out_specs=pl.BlockSpec((BS, D), lambda i: (i, 0)),
        out_shape=jax.ShapeDtypeStruct((S, D), jnp.float32),
    )(yt2, wcomb, dense, nw)


_SC_MESH = plsc.VectorSubcoreMesh(core_axis_name="c", subcore_axis_name="s")
_NC = 2
_NW = 32


NBUF = NSLOT + 8
_TPW = S // _NW


@functools.partial(
    pl.kernel,
    mesh=_SC_MESH,
    out_type=jax.ShapeDtypeStruct((NBUF, D), jnp.float32),
    scratch_types=[
        pltpu.VMEM((_TPW, D), jnp.float32),
        pltpu.VMEM((_TPW,), jnp.int32),
        pltpu.VMEM((_TPW,), jnp.int32),
        pltpu.SemaphoreType.DMA,
        pltpu.SemaphoreType.DMA,
    ],
)
def _sc_dispatch(xf_hbm, idxa_hbm, idxb_hbm, buf_hbm, rows_v, ia_v, ib_v, sa, sb):
    wid = lax.axis_index("s") * _NC + lax.axis_index("c")
    base = wid * _TPW
    pltpu.sync_copy(idxa_hbm.at[pl.ds(base, _TPW)], ia_v)
    pltpu.sync_copy(idxb_hbm.at[pl.ds(base, _TPW)], ib_v)
    pltpu.sync_copy(xf_hbm.at[pl.ds(base, _TPW)], rows_v)
    cpa = pltpu.async_copy(rows_v, buf_hbm.at[ia_v], sa)
    cpb = pltpu.async_copy(rows_v, buf_hbm.at[ib_v], sb)
    cpa.wait()
    cpb.wait()


def _make_sc_gather(n_rows):
    per_w = n_rows // _NW
    chunk = 32
    n_chunks = per_w // chunk

    @functools.partial(
        pl.kernel,
        mesh=_SC_MESH,
        out_type=jax.ShapeDtypeStruct((n_rows, D), jnp.float32),
        scratch_types=[pltpu.VMEM((per_w,), jnp.int32)]
        + [pltpu.VMEM((chunk, D), jnp.float32)] * 2
        + [pltpu.SemaphoreType.DMA] * 2,
    )
    def gather(table_hbm, idx_hbm, out_hbm, idx_v, buf0, buf1, sem0, sem1):
        bufs = (buf0, buf1)
        sems = (sem0, sem1)
        wid = lax.axis_index("s") * _NC + lax.axis_index("c")
        base_w = wid * per_w
        pltpu.sync_copy(idx_hbm.at[pl.ds(base_w, per_w)], idx_v)

        def start(c):
            return pltpu.async_copy(
                table_hbm.at[idx_v.at[pl.ds(c * chunk, chunk)]],
                bufs[c % 2], sems[c % 2])

        cps = [start(0), None]
        for c in range(n_chunks):
            if c + 1 < n_chunks:
                cps[(c + 1) % 2] = start(c + 1)
            cps[c % 2].wait()
            pltpu.sync_copy(bufs[c % 2], out_hbm.at[pl.ds(base_w + c * chunk, chunk)])

    return gather


_sc_gather_combine = _make_sc_gather(TK)


def kernel(x, wq, wk, wv, wo, attn_norm_w, w1, w2, w3, ffn_norm_w,
           wr, we1, we2, moe_norm_w):
    xf = x.reshape(S, D)

    wr_pad = jnp.pad(wr, ((0, 0), (0, LANES - E)))
    q, k, v, oh, meta = _qkv_route(xf, wq, wk, wv, wr_pad)
    qh = q.reshape(S, H, DH).transpose(1, 0, 2)
    kth = k.reshape(S, H, DH).transpose(1, 2, 0)
    vh = v.reshape(S, H, DH).transpose(1, 0, 2)
    o = _attn(qh, kth, vh).transpose(1, 0, 2).reshape(S, D)
    h = _oproj(o, wo, xf, attn_norm_w.reshape(1, D))
    dense = _ffn(h, w1, w3, w2, ffn_norm_w.reshape(1, D))

    idxk, wcomb, cnts = _route_pos(oh, meta)
    gath_idx = idxk[:, K:2 * K].reshape(TK)
    counts = cnts[0, :E]

    buf = _sc_dispatch(xf, idxk[:, 0], idxk[:, 1])
    eout = _experts(counts, buf, we1, we2)
    yt = _sc_gather_combine(eout, gath_idx)

    out = _combine(yt.reshape(S, 2 * D), wcomb, dense, moe_norm_w.reshape(1, D))
    return out.reshape(1, S, D)


# speedup vs baseline: 1.0434x; 1.0248x over previous
import functools

import jax
import jax.numpy as jnp
from jax import lax
from jax.experimental import pallas as pl
from jax.experimental.pallas import tpu as pltpu
from jax.experimental.pallas import tpu_sc as plsc

S, D, H, E, K, DFF, CAP = 2048, 1024, 16, 8, 2, 2048, 768
DH = D // H
TK = S * K
NSLOT = E * CAP
BS = 256
EBS = 256
EPS = 1e-5
LANES = 128


def _qkv_route_body(x_ref, wq_ref, wk_ref, wv_ref, wr_ref,
                    q_ref, k_ref, v_ref, oh_ref, meta_ref):
    xf32 = x_ref[...]
    x = xf32.astype(jnp.bfloat16)
    wqb = wq_ref[...].astype(jnp.bfloat16)
    wkb = wk_ref[...].astype(jnp.bfloat16)
    wvb = wv_ref[...].astype(jnp.bfloat16)
    q_ref[...] = jnp.dot(x, wqb, preferred_element_type=jnp.float32)
    k_ref[...] = jnp.dot(x, wkb, preferred_element_type=jnp.float32)
    v_ref[...] = jnp.dot(x, wvb, preferred_element_type=jnp.float32)
    logits = jnp.dot(xf32, wr_ref[...], preferred_element_type=jnp.float32)
    lane = jax.lax.broadcasted_iota(jnp.int32, (BS, LANES), 1)
    l = jnp.where(lane < E, logits, jnp.float32(-1e30))
    m = jnp.max(l, axis=1, keepdims=True)
    ex = jnp.exp(l - m)
    p = ex / jnp.sum(ex, axis=1, keepdims=True)
    m1 = jnp.max(p, axis=1, keepdims=True)
    i1 = jnp.min(jnp.where(p == m1, lane, jnp.int32(LANES)), axis=1, keepdims=True)
    p2 = jnp.where(lane == i1, jnp.float32(-1.0), p)
    m2 = jnp.max(p2, axis=1, keepdims=True)
    i2 = jnp.min(jnp.where(p2 == m2, lane, jnp.int32(LANES)), axis=1, keepdims=True)
    den = m1 + m2
    w0 = m1 / den
    w1 = m2 / den
    oh_ref[...] = ((lane == i1) | (lane == i2)).astype(jnp.bfloat16)
    i1f = i1.astype(jnp.float32)
    i2f = i2.astype(jnp.float32)
    meta_ref[...] = jnp.where(lane == 0, i1f,
                    jnp.where(lane == 1, i2f,
                    jnp.where(lane == 2, w0,
                    jnp.where(lane == 3, w1, jnp.float32(0.0)))))


def _qkv_route(xf, wq, wk, wv, wr_pad):
    w_spec = pl.BlockSpec((D, D), lambda i: (0, 0))
    r_spec = pl.BlockSpec((BS, D), lambda i: (i, 0))
    l_spec = pl.BlockSpec((BS, LANES), lambda i: (i, 0))
    return pl.pallas_call(
        _qkv_route_body,
        grid=(S // BS,),
        in_specs=[r_spec, w_spec, w_spec, w_spec,
                  pl.BlockSpec((D, LANES), lambda i: (0, 0))],
        out_specs=[r_spec, r_spec, r_spec, l_spec, l_spec],
        out_shape=[jax.ShapeDtypeStruct((S, D), jnp.float32)] * 3
        + [jax.ShapeDtypeStruct((S, LANES), jnp.bfloat16),
           jax.ShapeDtypeStruct((S, LANES), jnp.float32)],
    )(xf, wq, wk, wv, wr_pad)


def _make_attn(sk, qoff, nqb):
    def body(q_ref, kt_ref, v_ref, o_ref):
        qb = pl.program_id(1) + qoff
        qb16 = q_ref[0].astype(jnp.bfloat16)
        kt16 = kt_ref[0].astype(jnp.bfloat16)
        s = jnp.dot(qb16, kt16, preferred_element_type=jnp.float32)
        s = s * 0.125
        row = jax.lax.broadcasted_iota(jnp.int32, (BS, sk), 0) + qb * BS
        col = jax.lax.broadcasted_iota(jnp.int32, (BS, sk), 1)
        s = jnp.where(col <= row, s, jnp.float32(-1e30))
        m = jnp.max(s, axis=1, keepdims=True)
        e = jnp.exp(s - m)
        p = (e / jnp.sum(e, axis=1, keepdims=True)).astype(jnp.bfloat16)
        v16 = v_ref[0].astype(jnp.bfloat16)
        o_ref[0] = jnp.dot(p, v16, preferred_element_type=jnp.float32)

    def call(qh, kth, vh):
        return pl.pallas_call(
            body,
            grid=(H, nqb),
            in_specs=[
                pl.BlockSpec((1, BS, DH), lambda h, qb: (h, qb + qoff, 0)),
                pl.BlockSpec((1, DH, sk), lambda h, qb: (h, 0, 0)),
                pl.BlockSpec((1, sk, DH), lambda h, qb: (h, 0, 0)),
            ],
            out_specs=pl.BlockSpec((1, BS, DH), lambda h, qb: (h, qb, 0)),
            out_shape=jax.ShapeDtypeStruct((H, nqb * BS, DH), jnp.float32),
        )(qh, kth, vh)

    return call


_attn_lo = _make_attn(S // 2, 0, S // BS // 2)
_attn_hi = _make_attn(S, S // BS // 2, S // BS // 2)


def _attn(qh, kth, vh):
    return jnp.concatenate([_attn_lo(qh, kth, vh), _attn_hi(qh, kth, vh)], axis=1)


def _rmsnorm(v, w_row):
    ms = jnp.mean(v * v, axis=1, keepdims=True)
    return v * jax.lax.rsqrt(ms + EPS) * w_row


def _oproj_body(o_ref, wo_ref, x_ref, nw_ref, h_ref):
    ob = o_ref[...].astype(jnp.bfloat16)
    wob = wo_ref[...].astype(jnp.bfloat16)
    prj = jnp.dot(ob, wob, preferred_element_type=jnp.float32)
    h_ref[...] = x_ref[...] + _rmsnorm(prj, nw_ref[...])


def _oproj(o, wo, xf, nw):
    return pl.pallas_call(
        _oproj_body,
        grid=(S // BS,),
        in_specs=[
            pl.BlockSpec((BS, D), lambda i: (i, 0)),
            pl.BlockSpec((D, D), lambda i: (0, 0)),
            pl.BlockSpec((BS, D), lambda i: (i, 0)),
            pl.BlockSpec((1, D), lambda i: (0, 0)),
        ],
        out_specs=pl.BlockSpec((BS, D), lambda i: (i, 0)),
        out_shape=jax.ShapeDtypeStruct((S, D), jnp.float32),
    )(o, wo, xf, nw)


def _ffn_body(h_ref, w1_ref, w3_ref, w2_ref, nw_ref, out_ref):
    h = h_ref[...]
    hb = h.astype(jnp.bfloat16)
    a = jnp.dot(hb, w1_ref[...].astype(jnp.bfloat16), preferred_element_type=jnp.float32)
    b = jnp.dot(hb, w3_ref[...].astype(jnp.bfloat16), preferred_element_type=jnp.float32)
    g = ((a / (1.0 + jnp.exp(-a))) * b).astype(jnp.bfloat16)
    ff = jnp.dot(g, w2_ref[...].astype(jnp.bfloat16), preferred_element_type=jnp.float32)
    out_ref[...] = h + _rmsnorm(ff, nw_ref[...])


def _ffn(h, w1, w3, w2, nw):
    return pl.pallas_call(
        _ffn_body,
        grid=(S // BS,),
        in_specs=[
            pl.BlockSpec((BS, D), lambda i: (i, 0)),
            pl.BlockSpec((D, DFF), lambda i: (0, 0)),
            pl.BlockSpec((D, DFF), lambda i: (0, 0)),
            pl.BlockSpec((DFF, D), lambda i: (0, 0)),
            pl.BlockSpec((1, D), lambda i: (0, 0)),
        ],
        out_specs=pl.BlockSpec((BS, D), lambda i: (i, 0)),
        out_shape=jax.ShapeDtypeStruct((S, D), jnp.float32),
    )(h, w1, w3, w2, nw)


def _route_pos_body(oh_ref, meta_ref, idx_ref, w_ref, cnt_ref):
    rb = pl.program_id(0)
    row = jax.lax.broadcasted_iota(jnp.int32, (BS, S), 0) + rb * BS
    col = jax.lax.broadcasted_iota(jnp.int32, (BS, S), 1)
    ltri = (col < row).astype(jnp.bfloat16)
    sx = jnp.dot(ltri, oh_ref[...], preferred_element_type=jnp.float32)
    lane = jax.lax.broadcasted_iota(jnp.int32, (BS, LANES), 1)
    lanef = lane.astype(jnp.float32)
    i1f = meta_ref[:, 0:1]
    i2f = meta_ref[:, 1:2]
    w0 = meta_ref[:, 2:3]
    w1 = meta_ref[:, 3:4]
    pos1 = jnp.sum(jnp.where(lanef == i1f, sx, 0.0), axis=1, keepdims=True).astype(jnp.int32)
    pos2 = jnp.sum(jnp.where(lanef == i2f, sx, 0.0), axis=1, keepdims=True).astype(jnp.int32)
    k1 = pos1 < CAP
    k2 = pos2 < CAP
    g1 = i1f.astype(jnp.int32) * CAP + jnp.minimum(pos1, CAP - 1)
    g2 = i2f.astype(jnp.int32) * CAP + jnp.minimum(pos2, CAP - 1)
    s1 = jnp.where(k1, g1, NSLOT)
    s2 = jnp.where(k2, g2, NSLOT)
    idx = jnp.where(lane == 0, s1,
          jnp.where(lane == 1, s2,
          jnp.where(lane == 2, g1,
          jnp.where(lane == 3, g2, jnp.int32(0)))))
    w = jnp.where(lane == 0, w0 * k1.astype(jnp.float32),
        jnp.where(lane == 1, w1 * k2.astype(jnp.float32), jnp.float32(0.0)))
    idx_ref[...] = idx
    w_ref[...] = w

    @pl.when(rb == S // BS - 1)
    def _():
        tot = sx[BS - 1:BS, :] + oh_ref[S - 1:S, :].astype(jnp.float32)
        cnt_ref[...] = jnp.broadcast_to(tot, (8, LANES)).astype(jnp.int32)


def _route_pos(oh, meta):
    return pl.pallas_call(
        _route_pos_body,
        grid=(S // BS,),
        in_specs=[
            pl.BlockSpec((S, LANES), lambda i: (0, 0)),
            pl.BlockSpec((BS, LANES), lambda i: (i, 0)),
        ],
        out_specs=[
            pl.BlockSpec((BS, LANES), lambda i: (i, 0)),
            pl.BlockSpec((BS, LANES), lambda i: (i, 0)),
            pl.BlockSpec((8, LANES), lambda i: (0, 0)),
        ],
        out_shape=[
            jax.ShapeDtypeStruct((S, LANES), jnp.int32),
            jax.ShapeDtypeStruct((S, LANES), jnp.float32),
            jax.ShapeDtypeStruct((8, LANES), jnp.int32),
        ],
    )(oh, meta)


def _experts_flat_body(cnt_ref, b_ref, w1_ref, w2_ref, o_ref):
    e = pl.program_id(0)
    c = pl.program_id(1)

    @pl.when(c * EBS < cnt_ref[e])
    def _():
        x = b_ref[...].astype(jnp.bfloat16)
        a = jnp.dot(x, w1_ref[0].astype(jnp.bfloat16), preferred_element_type=jnp.float32)
        hdn = jax.nn.gelu(a).astype(jnp.bfloat16)
        o_ref[...] = jnp.dot(hdn, w2_ref[0].astype(jnp.bfloat16),
                             preferred_element_type=jnp.float32)


def _experts(counts, buf_flat, we1, we2):
    return pl.pallas_call(
        _experts_flat_body,
        grid=(E, CAP // EBS),
        in_specs=[
            pl.BlockSpec(memory_space=pltpu.SMEM),
            pl.BlockSpec((EBS, D), lambda e, c: (e * (CAP // EBS) + c, 0)),
            pl.BlockSpec((1, D, DFF), lambda e, c: (e, 0, 0)),
            pl.BlockSpec((1, DFF, D), lambda e, c: (e, 0, 0)),
        ],
        out_specs=pl.BlockSpec((EBS, D), lambda e, c: (e * (CAP // EBS) + c, 0)),
        out_shape=jax.ShapeDtypeStruct((NSLOT, D), jnp.float32),
    )(counts, buf_flat, we1, we2)


def _combine_body(yt_ref, w_ref, dense_ref, nw_ref, out_ref):
    y = yt_ref[:, :D] * w_ref[:, 0:1] + yt_ref[:, D:] * w_ref[:, 1:2]
    out_ref[...] = dense_ref[...] + _rmsnorm(y, nw_ref[...])


def _combine(yt2, wcomb, dense, nw):
    return pl.pallas_call(
        _combine_body,
        grid=(S // BS,),
        in_specs=[
            pl.BlockSpec((BS, 2 * D), lambda i: (i, 0)),
            pl.BlockSpec((BS, LANES), lambda i: (i, 0)),
            pl.BlockSpec((BS, D), lambda i: (i, 0)),
            pl.BlockSpec((1, D), lambda i: (0, 0)),
        ],
        out_specs=pl.BlockSpec((BS, D), lambda i: (i, 0)),
        out_shape=jax.ShapeDtypeStruct((S, D), jnp.float32),
    )(yt2, wcomb, dense, nw)


_SC_MESH = plsc.VectorSubcoreMesh(core_axis_name="c", subcore_axis_name="s")
_NC = 2
_NW = 32


NBUF = NSLOT + 8
_TPW = S // _NW


@functools.partial(
    pl.kernel,
    mesh=_SC_MESH,
    out_type=jax.ShapeDtypeStruct((NBUF, D), jnp.float32),
    scratch_types=[
        pltpu.VMEM((_TPW, D), jnp.float32),
        pltpu.VMEM((_TPW,), jnp.int32),
        pltpu.VMEM((_TPW,), jnp.int32),
        pltpu.SemaphoreType.DMA,
        pltpu.SemaphoreType.DMA,
    ],
)
def _sc_dispatch(xf_hbm, idxa_hbm, idxb_hbm, buf_hbm, rows_v, ia_v, ib_v, sa, sb):
    wid = lax.axis_index("s") * _NC + lax.axis_index("c")
    base = wid * _TPW
    pltpu.sync_copy(idxa_hbm.at[pl.ds(base, _TPW)], ia_v)
    pltpu.sync_copy(idxb_hbm.at[pl.ds(base, _TPW)], ib_v)
    pltpu.sync_copy(xf_hbm.at[pl.ds(base, _TPW)], rows_v)
    cpa = pltpu.async_copy(rows_v, buf_hbm.at[ia_v], sa)
    cpb = pltpu.async_copy(rows_v, buf_hbm.at[ib_v], sb)
    cpa.wait()
    cpb.wait()


def _make_sc_gather(n_rows):
    per_w = n_rows // _NW
    chunk = 32
    n_chunks = per_w // chunk

    @functools.partial(
        pl.kernel,
        mesh=_SC_MESH,
        out_type=jax.ShapeDtypeStruct((n_rows, D), jnp.float32),
        scratch_types=[pltpu.VMEM((per_w,), jnp.int32)]
        + [pltpu.VMEM((chunk, D), jnp.float32)] * 2
        + [pltpu.SemaphoreType.DMA] * 2,
    )
    def gather(table_hbm, idx_hbm, out_hbm, idx_v, buf0, buf1, sem0, sem1):
        bufs = (buf0, buf1)
        sems = (sem0, sem1)
        wid = lax.axis_index("s") * _NC + lax.axis_index("c")
        base_w = wid * per_w
        pltpu.sync_copy(idx_hbm.at[pl.ds(base_w, per_w)], idx_v)

        def start(c):
            return pltpu.async_copy(
                table_hbm.at[idx_v.at[pl.ds(c * chunk, chunk)]],
                bufs[c % 2], sems[c % 2])

        cps = [start(0), None]
        for c in range(n_chunks):
            if c + 1 < n_chunks:
                cps[(c + 1) % 2] = start(c + 1)
            cps[c % 2].wait()
            pltpu.sync_copy(bufs[c % 2], out_hbm.at[pl.ds(base_w + c * chunk, chunk)])

    return gather


_sc_gather_combine = _make_sc_gather(TK)


def kernel(x, wq, wk, wv, wo, attn_norm_w, w1, w2, w3, ffn_norm_w,
           wr, we1, we2, moe_norm_w):
    xf = x.reshape(S, D)

    wr_pad = jnp.pad(wr, ((0, 0), (0, LANES - E)))
    q, k, v, oh, meta = _qkv_route(xf, wq, wk, wv, wr_pad)
    qh = q.reshape(S, H, DH).transpose(1, 0, 2)
    kth = k.reshape(S, H, DH).transpose(1, 2, 0)
    vh = v.reshape(S, H, DH).transpose(1, 0, 2)
    o = _attn(qh, kth, vh).transpose(1, 0, 2).reshape(S, D)
    h = _oproj(o, wo, xf, attn_norm_w.reshape(1, D))
    dense = _ffn(h, w1, w3, w2, ffn_norm_w.reshape(1, D))

    idxk, wcomb, cnts = _route_pos(oh, meta)
    gath_idx = idxk[:, K:2 * K].reshape(TK)
    counts = cnts[0, :E]

    buf = _sc_dispatch(xf, idxk[:, 0], idxk[:, 1])
    eout = _experts(counts, buf, we1, we2)
    yt = _sc_gather_combine(eout, gath_idx)

    out = _combine(yt.reshape(S, 2 * D), wcomb, dense, moe_norm_w.reshape(1, D))
    return out.reshape(1, S, D)


# speedup vs baseline: 1.0468x; 1.0033x over previous
import functools

import jax
import jax.numpy as jnp
from jax import lax
from jax.experimental import pallas as pl
from jax.experimental.pallas import tpu as pltpu
from jax.experimental.pallas import tpu_sc as plsc

S, D, H, E, K, DFF, CAP = 2048, 1024, 16, 8, 2, 2048, 768
DH = D // H
TK = S * K
NSLOT = E * CAP
BS = 256
EBS = 256
EPS = 1e-5
LANES = 128


def _qkv_route_body(x_ref, wq_ref, wk_ref, wv_ref, wr_ref,
                    q_ref, k_ref, v_ref, oh_ref, meta_ref):
    xf32 = x_ref[...]
    x = xf32.astype(jnp.bfloat16)
    wqb = wq_ref[...].astype(jnp.bfloat16)
    wkb = wk_ref[...].astype(jnp.bfloat16)
    wvb = wv_ref[...].astype(jnp.bfloat16)
    q_ref[...] = jnp.dot(x, wqb, preferred_element_type=jnp.float32)
    k_ref[...] = jnp.dot(x, wkb, preferred_element_type=jnp.float32)
    v_ref[...] = jnp.dot(x, wvb, preferred_element_type=jnp.float32)
    logits = jnp.dot(xf32, wr_ref[...], preferred_element_type=jnp.float32)
    lane = jax.lax.broadcasted_iota(jnp.int32, (BS, LANES), 1)
    l = jnp.where(lane < E, logits, jnp.float32(-1e30))
    m = jnp.max(l, axis=1, keepdims=True)
    ex = jnp.exp(l - m)
    p = ex / jnp.sum(ex, axis=1, keepdims=True)
    m1 = jnp.max(p, axis=1, keepdims=True)
    i1 = jnp.min(jnp.where(p == m1, lane, jnp.int32(LANES)), axis=1, keepdims=True)
    p2 = jnp.where(lane == i1, jnp.float32(-1.0), p)
    m2 = jnp.max(p2, axis=1, keepdims=True)
    i2 = jnp.min(jnp.where(p2 == m2, lane, jnp.int32(LANES)), axis=1, keepdims=True)
    den = m1 + m2
    w0 = m1 / den
    w1 = m2 / den
    oh_ref[...] = ((lane == i1) | (lane == i2)).astype(jnp.bfloat16)
    i1f = i1.astype(jnp.float32)
    i2f = i2.astype(jnp.float32)
    meta_ref[...] = jnp.where(lane == 0, i1f,
                    jnp.where(lane == 1, i2f,
                    jnp.where(lane == 2, w0,
                    jnp.where(lane == 3, w1, jnp.float32(0.0)))))


def _qkv_route(xf, wq, wk, wv, wr_pad):
    w_spec = pl.BlockSpec((D, D), lambda i: (0, 0))
    r_spec = pl.BlockSpec((BS, D), lambda i: (i, 0))
    l_spec = pl.BlockSpec((BS, LANES), lambda i: (i, 0))
    return pl.pallas_call(
        _qkv_route_body,
        grid=(S // BS,),
        in_specs=[r_spec, w_spec, w_spec, w_spec,
                  pl.BlockSpec((D, LANES), lambda i: (0, 0))],
        out_specs=[r_spec, r_spec, r_spec, l_spec, l_spec],
        out_shape=[jax.ShapeDtypeStruct((S, D), jnp.float32)] * 3
        + [jax.ShapeDtypeStruct((S, LANES), jnp.bfloat16),
           jax.ShapeDtypeStruct((S, LANES), jnp.float32)],
    )(xf, wq, wk, wv, wr_pad)


def _make_attn(sk, qoff, nqb):
    def body(q_ref, kt_ref, v_ref, o_ref):
        qb = pl.program_id(1) + qoff
        qb16 = q_ref[0].astype(jnp.bfloat16)
        kt16 = kt_ref[0].astype(jnp.bfloat16)
        s = jnp.dot(qb16, kt16, preferred_element_type=jnp.float32)
        s = s * 0.125
        row = jax.lax.broadcasted_iota(jnp.int32, (BS, sk), 0) + qb * BS
        col = jax.lax.broadcasted_iota(jnp.int32, (BS, sk), 1)
        s = jnp.where(col <= row, s, jnp.float32(-1e30))
        m = jnp.max(s, axis=1, keepdims=True)
        e = jnp.exp(s - m)
        p = (e / jnp.sum(e, axis=1, keepdims=True)).astype(jnp.bfloat16)
        v16 = v_ref[0].astype(jnp.bfloat16)
        o_ref[0] = jnp.dot(p, v16, preferred_element_type=jnp.float32)

    def call(qh, kth, vh):
        return pl.pallas_call(
            body,
            grid=(H, nqb),
            in_specs=[
                pl.BlockSpec((1, BS, DH), lambda h, qb: (h, qb + qoff, 0)),
                pl.BlockSpec((1, DH, sk), lambda h, qb: (h, 0, 0)),
                pl.BlockSpec((1, sk, DH), lambda h, qb: (h, 0, 0)),
            ],
            out_specs=pl.BlockSpec((1, BS, DH), lambda h, qb: (h, qb, 0)),
            out_shape=jax.ShapeDtypeStruct((H, nqb * BS, DH), jnp.float32),
        )(qh, kth, vh)

    return call


_attn_parts = [_make_attn((j + 1) * S // 4, j * (S // BS) // 4, (S // BS) // 4)
               for j in range(4)]


def _attn(qh, kth, vh):
    return jnp.concatenate([p(qh, kth, vh) for p in _attn_parts], axis=1)


def _rmsnorm(v, w_row):
    ms = jnp.mean(v * v, axis=1, keepdims=True)
    return v * jax.lax.rsqrt(ms + EPS) * w_row


def _oproj_body(o_ref, wo_ref, x_ref, nw_ref, h_ref):
    ob = o_ref[...].astype(jnp.bfloat16)
    wob = wo_ref[...].astype(jnp.bfloat16)
    prj = jnp.dot(ob, wob, preferred_element_type=jnp.float32)
    h_ref[...] = x_ref[...] + _rmsnorm(prj, nw_ref[...])


def _oproj(o, wo, xf, nw):
    return pl.pallas_call(
        _oproj_body,
        grid=(S // BS,),
        in_specs=[
            pl.BlockSpec((BS, D), lambda i: (i, 0)),
            pl.BlockSpec((D, D), lambda i: (0, 0)),
            pl.BlockSpec((BS, D), lambda i: (i, 0)),
            pl.BlockSpec((1, D), lambda i: (0, 0)),
        ],
        out_specs=pl.BlockSpec((BS, D), lambda i: (i, 0)),
        out_shape=jax.ShapeDtypeStruct((S, D), jnp.float32),
    )(o, wo, xf, nw)


def _ffn_body(h_ref, w1_ref, w3_ref, w2_ref, nw_ref, out_ref):
    h = h_ref[...]
    hb = h.astype(jnp.bfloat16)
    a = jnp.dot(hb, w1_ref[...].astype(jnp.bfloat16), preferred_element_type=jnp.float32)
    b = jnp.dot(hb, w3_ref[...].astype(jnp.bfloat16), preferred_element_type=jnp.float32)
    g = ((a / (1.0 + jnp.exp(-a))) * b).astype(jnp.bfloat16)
    ff = jnp.dot(g, w2_ref[...].astype(jnp.bfloat16), preferred_element_type=jnp.float32)
    out_ref[...] = h + _rmsnorm(ff, nw_ref[...])


def _ffn(h, w1, w3, w2, nw):
    return pl.pallas_call(
        _ffn_body,
        grid=(S // BS,),
        in_specs=[
            pl.BlockSpec((BS, D), lambda i: (i, 0)),
            pl.BlockSpec((D, DFF), lambda i: (0, 0)),
            pl.BlockSpec((D, DFF), lambda i: (0, 0)),
            pl.BlockSpec((DFF, D), lambda i: (0, 0)),
            pl.BlockSpec((1, D), lambda i: (0, 0)),
        ],
        out_specs=pl.BlockSpec((BS, D), lambda i: (i, 0)),
        out_shape=jax.ShapeDtypeStruct((S, D), jnp.float32),
    )(h, w1, w3, w2, nw)


def _route_pos_body(oh_ref, meta_ref, idx_ref, w_ref, cnt_ref):
    rb = pl.program_id(0)
    row = jax.lax.broadcasted_iota(jnp.int32, (BS, S), 0) + rb * BS
    col = jax.lax.broadcasted_iota(jnp.int32, (BS, S), 1)
    ltri = (col < row).astype(jnp.bfloat16)
    sx = jnp.dot(ltri, oh_ref[...], preferred_element_type=jnp.float32)
    lane = jax.lax.broadcasted_iota(jnp.int32, (BS, LANES), 1)
    lanef = lane.astype(jnp.float32)
    i1f = meta_ref[:, 0:1]
    i2f = meta_ref[:, 1:2]
    w0 = meta_ref[:, 2:3]
    w1 = meta_ref[:, 3:4]
    pos1 = jnp.sum(jnp.where(lanef == i1f, sx, 0.0), axis=1, keepdims=True).astype(jnp.int32)
    pos2 = jnp.sum(jnp.where(lanef == i2f, sx, 0.0), axis=1, keepdims=True).astype(jnp.int32)
    k1 = pos1 < CAP
    k2 = pos2 < CAP
    g1 = i1f.astype(jnp.int32) * CAP + jnp.minimum(pos1, CAP - 1)
    g2 = i2f.astype(jnp.int32) * CAP + jnp.minimum(pos2, CAP - 1)
    s1 = jnp.where(k1, g1, NSLOT)
    s2 = jnp.where(k2, g2, NSLOT)
    idx = jnp.where(lane == 0, s1,
          jnp.where(lane == 1, s2,
          jnp.where(lane == 2, g1,
          jnp.where(lane == 3, g2, jnp.int32(0)))))
    w = jnp.where(lane == 0, w0 * k1.astype(jnp.float32),
        jnp.where(lane == 1, w1 * k2.astype(jnp.float32), jnp.float32(0.0)))
    idx_ref[...] = idx
    w_ref[...] = w

    @pl.when(rb == S // BS - 1)
    def _():
        tot = sx[BS - 1:BS, :] + oh_ref[S - 1:S, :].astype(jnp.float32)
        cnt_ref[...] = jnp.broadcast_to(tot, (8, LANES)).astype(jnp.int32)


def _route_pos(oh, meta):
    return pl.pallas_call(
        _route_pos_body,
        grid=(S // BS,),
        in_specs=[
            pl.BlockSpec((S, LANES), lambda i: (0, 0)),
            pl.BlockSpec((BS, LANES), lambda i: (i, 0)),
        ],
        out_specs=[
            pl.BlockSpec((BS, LANES), lambda i: (i, 0)),
            pl.BlockSpec((BS, LANES), lambda i: (i, 0)),
            pl.BlockSpec((8, LANES), lambda i: (0, 0)),
        ],
        out_shape=[
            jax.ShapeDtypeStruct((S, LANES), jnp.int32),
            jax.ShapeDtypeStruct((S, LANES), jnp.float32),
            jax.ShapeDtypeStruct((8, LANES), jnp.int32),
        ],
    )(oh, meta)


def _experts_flat_body(cnt_ref, b_ref, w1_ref, w2_ref, o_ref):
    e = pl.program_id(0)
    c = pl.program_id(1)

    @pl.when(c * EBS < cnt_ref[e])
    def _():
        x = b_ref[...].astype(jnp.bfloat16)
        a = jnp.dot(x, w1_ref[0].astype(jnp.bfloat16), preferred_element_type=jnp.float32)
        hdn = jax.nn.gelu(a).astype(jnp.bfloat16)
        o_ref[...] = jnp.dot(hdn, w2_ref[0].astype(jnp.bfloat16),
                             preferred_element_type=jnp.float32)


def _experts(counts, buf_flat, we1, we2):
    return pl.pallas_call(
        _experts_flat_body,
        grid=(E, CAP // EBS),
        in_specs=[
            pl.BlockSpec(memory_space=pltpu.SMEM),
            pl.BlockSpec((EBS, D), lambda e, c: (e * (CAP // EBS) + c, 0)),
            pl.BlockSpec((1, D, DFF), lambda e, c: (e, 0, 0)),
            pl.BlockSpec((1, DFF, D), lambda e, c: (e, 0, 0)),
        ],
        out_specs=pl.BlockSpec((EBS, D), lambda e, c: (e * (CAP // EBS) + c, 0)),
        out_shape=jax.ShapeDtypeStruct((NSLOT, D), jnp.float32),
    )(counts, buf_flat, we1, we2)


def _combine_body(yt_ref, w_ref, dense_ref, nw_ref, out_ref):
    y = yt_ref[:, :D] * w_ref[:, 0:1] + yt_ref[:, D:] * w_ref[:, 1:2]
    out_ref[...] = dense_ref[...] + _rmsnorm(y, nw_ref[...])


def _combine(yt2, wcomb, dense, nw):
    return pl.pallas_call(
        _combine_body,
        grid=(S // BS,),
        in_specs=[
            pl.BlockSpec((BS, 2 * D), lambda i: (i, 0)),
            pl.BlockSpec((BS, LANES), lambda i: (i, 0)),
            pl.BlockSpec((BS, D), lambda i: (i, 0)),
            pl.BlockSpec((1, D), lambda i: (0, 0)),
        ],
        out_specs=pl.BlockSpec((BS, D), lambda i: (i, 0)),
        out_shape=jax.ShapeDtypeStruct((S, D), jnp.float32),
    )(yt2, wcomb, dense, nw)


_SC_MESH = plsc.VectorSubcoreMesh(core_axis_name="c", subcore_axis_name="s")
_NC = 2
_NW = 32


NBUF = NSLOT + 8
_TPW = S // _NW


@functools.partial(
    pl.kernel,
    mesh=_SC_MESH,
    out_type=jax.ShapeDtypeStruct((NBUF, D), jnp.float32),
    scratch_types=[
        pltpu.VMEM((_TPW, D), jnp.float32),
        pltpu.VMEM((_TPW,), jnp.int32),
        pltpu.VMEM((_TPW,), jnp.int32),
        pltpu.SemaphoreType.DMA,
        pltpu.SemaphoreType.DMA,
    ],
)
def _sc_dispatch(xf_hbm, idxa_hbm, idxb_hbm, buf_hbm, rows_v, ia_v, ib_v, sa, sb):
    wid = lax.axis_index("s") * _NC + lax.axis_index("c")
    base = wid * _TPW
    pltpu.sync_copy(idxa_hbm.at[pl.ds(base, _TPW)], ia_v)
    pltpu.sync_copy(idxb_hbm.at[pl.ds(base, _TPW)], ib_v)
    pltpu.sync_copy(xf_hbm.at[pl.ds(base, _TPW)], rows_v)
    cpa = pltpu.async_copy(rows_v, buf_hbm.at[ia_v], sa)
    cpb = pltpu.async_copy(rows_v, buf_hbm.at[ib_v], sb)
    cpa.wait()
    cpb.wait()


def _make_sc_gather(n_rows):
    per_w = n_rows // _NW
    chunk = 32
    n_chunks = per_w // chunk

    @functools.partial(
        pl.kernel,
        mesh=_SC_MESH,
        out_type=jax.ShapeDtypeStruct((n_rows, D), jnp.float32),
        scratch_types=[pltpu.VMEM((per_w,), jnp.int32)]
        + [pltpu.VMEM((chunk, D), jnp.float32)] * 2
        + [pltpu.SemaphoreType.DMA] * 2,
    )
    def gather(table_hbm, idx_hbm, out_hbm, idx_v, buf0, buf1, sem0, sem1):
        bufs = (buf0, buf1)
        sems = (sem0, sem1)
        wid = lax.axis_index("s") * _NC + lax.axis_index("c")
        base_w = wid * per_w
        pltpu.sync_copy(idx_hbm.at[pl.ds(base_w, per_w)], idx_v)

        def start(c):
            return pltpu.async_copy(
                table_hbm.at[idx_v.at[pl.ds(c * chunk, chunk)]],
                bufs[c % 2], sems[c % 2])

        cps = [start(0), None]
        for c in range(n_chunks):
            if c + 1 < n_chunks:
                cps[(c + 1) % 2] = start(c + 1)
            cps[c % 2].wait()
            pltpu.sync_copy(bufs[c % 2], out_hbm.at[pl.ds(base_w + c * chunk, chunk)])

    return gather


_sc_gather_combine = _make_sc_gather(TK)


def kernel(x, wq, wk, wv, wo, attn_norm_w, w1, w2, w3, ffn_norm_w,
           wr, we1, we2, moe_norm_w):
    xf = x.reshape(S, D)

    wr_pad = jnp.pad(wr, ((0, 0), (0, LANES - E)))
    q, k, v, oh, meta = _qkv_route(xf, wq, wk, wv, wr_pad)
    qh = q.reshape(S, H, DH).transpose(1, 0, 2)
    kth = k.reshape(S, H, DH).transpose(1, 2, 0)
    vh = v.reshape(S, H, DH).transpose(1, 0, 2)
    o = _attn(qh, kth, vh).transpose(1, 0, 2).reshape(S, D)
    h = _oproj(o, wo, xf, attn_norm_w.reshape(1, D))
    dense = _ffn(h, w1, w3, w2, ffn_norm_w.reshape(1, D))

    idxk, wcomb, cnts = _route_pos(oh, meta)
    gath_idx = idxk[:, K:2 * K].reshape(TK)
    counts = cnts[0, :E]

    buf = _sc_dispatch(xf, idxk[:, 0], idxk[:, 1])
    eout = _experts(counts, buf, we1, we2)
    yt = _sc_gather_combine(eout, gath_idx)

    out = _combine(yt.reshape(S, 2 * D), wcomb, dense, moe_norm_w.reshape(1, D))
    return out.reshape(1, S, D)
